# Initial kernel scaffold; baseline (speedup 1.0000x reference)
import jax
import jax.numpy as jnp
from jax.experimental import pallas as pl


def kernel(x_bc1t, topk_idx, topk_weights, gate_up_all, down_all):
    raise NotImplementedError("write your pallas kernel here")


# speedup vs baseline: 9.8360x; 9.8360x over previous
import jax
import jax.numpy as jnp
from jax.experimental import pallas as pl
from jax.experimental.pallas import tpu as pltpu

EXPERT_INTER = 1408
HIDDEN = 2048
TOP_K = 8

RB1 = 128
RB2 = 512


def _inter_kernel(idx_ref, w_ref, x_ref, gate_ref, up_ref, o_ref):
    k = pl.program_id(0)
    g = jax.lax.dot_general(
        x_ref[...], gate_ref[0],
        (((1,), (1,)), ((), ())),
        preferred_element_type=jnp.float32,
    )
    u = jax.lax.dot_general(
        x_ref[...], up_ref[0],
        (((1,), (1,)), ((), ())),
        preferred_element_type=jnp.float32,
    )
    o_ref[0] = (g * jax.nn.sigmoid(g)) * u * w_ref[k]


def _down_kernel(idx_ref, w_ref, inter_ref, down_ref, o_ref):
    k = pl.program_id(1)
    part = jax.lax.dot_general(
        inter_ref[0], down_ref[0],
        (((1,), (1,)), ((), ())),
        preferred_element_type=jnp.float32,
    )

    @pl.when(k == 0)
    def _init():
        o_ref[...] = part

    @pl.when(k > 0)
    def _acc():
        o_ref[...] += part


def kernel(x_bc1t, topk_idx, topk_weights, gate_up_all, down_all):
    x = x_bc1t.reshape(1, HIDDEN)
    idx = topk_idx.astype(jnp.int32)
    nb1 = EXPERT_INTER // RB1
    nb2 = HIDDEN // RB2

    inter = pl.pallas_call(
        _inter_kernel,
        grid_spec=pltpu.PrefetchScalarGridSpec(
            num_scalar_prefetch=2,
            grid=(TOP_K, nb1),
            in_specs=[
                pl.BlockSpec((1, HIDDEN), lambda k, b, idx, w: (0, 0)),
                pl.BlockSpec((1, RB1, HIDDEN),
                             lambda k, b, idx, w: (idx[k], b, 0)),
                pl.BlockSpec((1, RB1, HIDDEN),
                             lambda k, b, idx, w: (idx[k], b + EXPERT_INTER // RB1, 0)),
            ],
            out_specs=pl.BlockSpec((1, 1, RB1), lambda k, b, idx, w: (k, 0, b)),
        ),
        out_shape=jax.ShapeDtypeStruct((TOP_K, 1, EXPERT_INTER), jnp.float32),
    )(idx, topk_weights, x, gate_up_all, gate_up_all)

    out = pl.pallas_call(
        _down_kernel,
        grid_spec=pltpu.PrefetchScalarGridSpec(
            num_scalar_prefetch=2,
            grid=(nb2, TOP_K),
            in_specs=[
                pl.BlockSpec((1, 1, EXPERT_INTER), lambda b, k, idx, w: (k, 0, 0)),
                pl.BlockSpec((1, RB2, EXPERT_INTER),
                             lambda b, k, idx, w: (idx[k], b, 0)),
            ],
            out_specs=pl.BlockSpec((1, RB2), lambda b, k, idx, w: (0, b)),
        ),
        out_shape=jax.ShapeDtypeStruct((1, HIDDEN), jnp.float32),
    )(idx, topk_weights, inter, down_all)

    return out.reshape(1, HIDDEN, 1, 1)


# speedup vs baseline: 14.7835x; 1.5030x over previous
import jax
import jax.numpy as jnp
from jax.experimental import pallas as pl
from jax.experimental.pallas import tpu as pltpu

EXPERT_INTER = 1408
HIDDEN = 2048
TOP_K = 8

RB1 = 1408
RB2 = 2048


def _inter_kernel(idx_ref, w_ref, x_ref, gate_ref, up_ref, o_ref):
    k = pl.program_id(0)
    g = jax.lax.dot_general(
        x_ref[...], gate_ref[0],
        (((1,), (1,)), ((), ())),
        preferred_element_type=jnp.float32,
    )
    u = jax.lax.dot_general(
        x_ref[...], up_ref[0],
        (((1,), (1,)), ((), ())),
        preferred_element_type=jnp.float32,
    )
    o_ref[0] = (g * jax.nn.sigmoid(g)) * u * w_ref[k]


def _down_kernel(idx_ref, w_ref, inter_ref, down_ref, o_ref):
    k = pl.program_id(1)
    part = jax.lax.dot_general(
        inter_ref[0], down_ref[0],
        (((1,), (1,)), ((), ())),
        preferred_element_type=jnp.float32,
    )

    @pl.when(k == 0)
    def _init():
        o_ref[...] = part

    @pl.when(k > 0)
    def _acc():
        o_ref[...] += part


def kernel(x_bc1t, topk_idx, topk_weights, gate_up_all, down_all):
    x = x_bc1t.reshape(1, HIDDEN)
    idx = topk_idx.astype(jnp.int32)
    nb1 = EXPERT_INTER // RB1
    nb2 = HIDDEN // RB2

    inter = pl.pallas_call(
        _inter_kernel,
        grid_spec=pltpu.PrefetchScalarGridSpec(
            num_scalar_prefetch=2,
            grid=(TOP_K, nb1),
            in_specs=[
                pl.BlockSpec((1, HIDDEN), lambda k, b, idx, w: (0, 0)),
                pl.BlockSpec((1, RB1, HIDDEN),
                             lambda k, b, idx, w: (idx[k], b, 0)),
                pl.BlockSpec((1, RB1, HIDDEN),
                             lambda k, b, idx, w: (idx[k], b + EXPERT_INTER // RB1, 0)),
            ],
            out_specs=pl.BlockSpec((1, 1, RB1), lambda k, b, idx, w: (k, 0, b)),
        ),
        out_shape=jax.ShapeDtypeStruct((TOP_K, 1, EXPERT_INTER), jnp.float32),
    )(idx, topk_weights, x, gate_up_all, gate_up_all)

    out = pl.pallas_call(
        _down_kernel,
        grid_spec=pltpu.PrefetchScalarGridSpec(
            num_scalar_prefetch=2,
            grid=(nb2, TOP_K),
            in_specs=[
                pl.BlockSpec((1, 1, EXPERT_INTER), lambda b, k, idx, w: (k, 0, 0)),
                pl.BlockSpec((1, RB2, EXPERT_INTER),
                             lambda b, k, idx, w: (idx[k], b, 0)),
            ],
            out_specs=pl.BlockSpec((1, RB2), lambda b, k, idx, w: (0, b)),
        ),
        out_shape=jax.ShapeDtypeStruct((1, HIDDEN), jnp.float32),
    )(idx, topk_weights, inter, down_all)

    return out.reshape(1, HIDDEN, 1, 1)
